# Initial kernel scaffold; baseline (speedup 1.0000x reference)
import jax
import jax.numpy as jnp
from jax.experimental import pallas as pl


def kernel(x, gate_w, gate_b, w1, b1, w2, b2, w3, b3):
    raise NotImplementedError("write your pallas kernel here")


# speedup vs baseline: 2.8541x; 2.8541x over previous
import functools

import jax
import jax.numpy as jnp
from jax.experimental import pallas as pl
from jax.experimental.pallas import tpu as pltpu

D_MODEL = 768
D_FF = 2048
N_EXP = 8
K = 2
T = 2048
F_BLK = 512
N_FB = D_FF // F_BLK


def _moe_body(x_ref, gw_ref, gb_ref, w1_ref, b1_ref, w2_ref, b2_ref,
              w3_ref, b3_ref, out_ref, cw_ref):
    e = pl.program_id(0)
    fb = pl.program_id(1)

    @pl.when(jnp.logical_and(e == 0, fb == 0))
    def _gating():
        x = x_ref[...]
        logits = jax.lax.dot_general(
            x, gw_ref[...], (((1,), (1,)), ((), ())),
            preferred_element_type=jnp.float32) + gb_ref[...]
        m = jnp.max(logits, axis=-1, keepdims=True)
        ex = jnp.exp(logits - m)
        probs = ex / jnp.sum(ex, axis=-1, keepdims=True)
        iota = jax.lax.broadcasted_iota(jnp.int32, (T, N_EXP), 1)
        m1 = jnp.max(probs, axis=-1, keepdims=True)
        i1 = jnp.min(jnp.where(probs == m1, iota, N_EXP), axis=-1,
                     keepdims=True)
        probs2 = jnp.where(iota == i1, -1.0, probs)
        m2 = jnp.max(probs2, axis=-1, keepdims=True)
        i2 = jnp.min(jnp.where(probs2 == m2, iota, N_EXP), axis=-1,
                     keepdims=True)
        denom = m1 + m2 + 1e-6
        cw = (jnp.where(iota == i1, m1 / denom, 0.0)
              + jnp.where(iota == i2, m2 / denom, 0.0))
        cw_ref[...] = cw

    x = x_ref[...]
    lane = jax.lax.broadcasted_iota(jnp.int32, (T, N_EXP), 1)
    w = jnp.sum(jnp.where(lane == e, cw_ref[...], 0.0), axis=-1,
                keepdims=True)
    xw1 = jax.lax.dot_general(x, w1_ref[...], (((1,), (1,)), ((), ())),
                              preferred_element_type=jnp.float32)
    xw3 = jax.lax.dot_general(x, w3_ref[...], (((1,), (1,)), ((), ())),
                              preferred_element_type=jnp.float32)
    g = xw1 + b1_ref[...]
    h = g * jax.lax.logistic(g) * (xw3 + b3_ref[...])
    yp = jax.lax.dot_general(h, w2_ref[...], (((1,), (1,)), ((), ())),
                             preferred_element_type=jnp.float32)

    @pl.when(jnp.logical_and(e == 0, fb == 0))
    def _init():
        out_ref[...] = jnp.zeros_like(out_ref)

    contrib = w * yp
    @pl.when(fb == 0)
    def _bias():
        out_ref[...] += w * b2_ref[...] + contrib

    @pl.when(fb != 0)
    def _acc():
        out_ref[...] += contrib


def kernel(x, gate_w, gate_b, w1, b1, w2, b2, w3, b3):
    grid = (N_EXP, N_FB)
    return pl.pallas_call(
        _moe_body,
        grid=grid,
        in_specs=[
            pl.BlockSpec((T, D_MODEL), lambda e, fb: (0, 0)),
            pl.BlockSpec((N_EXP, D_MODEL), lambda e, fb: (0, 0)),
            pl.BlockSpec((1, N_EXP), lambda e, fb: (0, 0)),
            pl.BlockSpec((None, F_BLK, D_MODEL), lambda e, fb: (e, fb, 0)),
            pl.BlockSpec((None, 1, F_BLK), lambda e, fb: (e, 0, fb)),
            pl.BlockSpec((None, D_MODEL, F_BLK), lambda e, fb: (e, 0, fb)),
            pl.BlockSpec((None, 1, D_MODEL), lambda e, fb: (e, 0, 0)),
            pl.BlockSpec((None, F_BLK, D_MODEL), lambda e, fb: (e, fb, 0)),
            pl.BlockSpec((None, 1, F_BLK), lambda e, fb: (e, 0, fb)),
        ],
        out_specs=pl.BlockSpec((T, D_MODEL), lambda e, fb: (0, 0)),
        out_shape=jax.ShapeDtypeStruct((T, D_MODEL), jnp.float32),
        scratch_shapes=[pltpu.VMEM((T, N_EXP), jnp.float32)],
    )(x, gate_w, gate_b.reshape(1, N_EXP), w1, b1.reshape(N_EXP, 1, D_FF),
      w2, b2.reshape(N_EXP, 1, D_MODEL), w3, b3.reshape(N_EXP, 1, D_FF))
